# baseline (speedup 1.0000x reference)
import functools

import jax
import jax.numpy as jnp
from jax import lax
from jax.experimental import pallas as pl
from jax.experimental.pallas import tpu as pltpu
from jax.experimental.pallas import tpu_sc as plsc

N = 10000
NP = 10240
E = 320000
EP = 327680
IDX_ROWS = EP // 128
NC = 2
NS = 16
BR = 1024

_mesh = plsc.VectorSubcoreMesh(core_axis_name="c", subcore_axis_name="s")


def _zero_rows(buf, nrows, ncols):
    @pl.loop(0, nrows)
    def _(r):
        @pl.loop(0, ncols, step=16)
        def _(cc):
            buf[r, pl.ds(cc, 16)] = jnp.zeros((16,), jnp.float32)


_CH = 40


def _mp_loop(h_hbm, src_ref, dst_ref, base, rows_per, acc,
             sidx, didx, ra, rb, sa, sb, ssa, ssb):
    @pl.loop(0, rows_per, step=_CH)
    def _(ch):
        pltpu.sync_copy(src_ref.at[pl.ds(base + ch, _CH)], sidx)
        pltpu.sync_copy(dst_ref.at[pl.ds(base + ch, _CH)], didx)
        pltpu.async_copy(h_hbm.at[sidx.at[0]], ra, sa)
        pltpu.async_copy(h_hbm.at[sidx.at[1]], rb, sb)

        @pl.loop(0, _CH, step=2)
        def _(jj):
            pltpu.make_async_copy(h_hbm.at[sidx.at[jj]], ra, sa).wait()
            pltpu.async_copy(ra, acc.at[didx.at[jj]], ssa, add=True)
            pltpu.make_async_copy(h_hbm.at[sidx.at[jj + 1]], rb, sb).wait()
            pltpu.async_copy(rb, acc.at[didx.at[jj + 1]], ssb, add=True)
            pltpu.make_async_copy(ra, acc.at[didx.at[jj]], ssa).wait()

            @pl.when(jj + 2 < _CH)
            def _():
                pltpu.async_copy(h_hbm.at[sidx.at[jj + 2]], ra, sa)

            pltpu.make_async_copy(rb, acc.at[didx.at[jj + 1]], ssb).wait()

            @pl.when(jj + 3 < _CH)
            def _():
                pltpu.async_copy(h_hbm.at[sidx.at[jj + 3]], rb, sb)


def _sc_degrees(idx):
    rows_per_sub = IDX_ROWS // NS

    @functools.partial(
        pl.kernel,
        out_type=jax.ShapeDtypeStruct((2, NP, 128), jnp.float32),
        mesh=_mesh,
        scratch_types=[
            pltpu.VMEM_SHARED((NP, 128), jnp.float32),
            pltpu.VMEM((128, 128), jnp.float32),
            pltpu.VMEM((16, 128), jnp.int32),
        ],
    )
    def k(idx_hbm, out_hbm, acc, ones, didx):
        c = lax.axis_index("c")
        s = lax.axis_index("s")
        _zero_rows(ones, 128, 128)
        for t in range(5):
            pltpu.sync_copy(ones, acc.at[pl.ds(s * 640 + t * 128, 128)])

        @pl.loop(0, 128)
        def _(r):
            @pl.loop(0, 128, step=16)
            def _(cc):
                ones[r, pl.ds(cc, 16)] = jnp.ones((16,), jnp.float32)

        plsc.subcore_barrier()

        @pl.loop(0, rows_per_sub, step=16)
        def _(ch):
            pltpu.sync_copy(
                idx_hbm.at[c].at[pl.ds(s * rows_per_sub + ch, 16)], didx)

            @pl.loop(0, 16)
            def _(j):
                pltpu.sync_copy(ones, acc.at[didx.at[j]], add=True)

        plsc.subcore_barrier()
        pltpu.sync_copy(acc.at[pl.ds(s * 640, 640)],
                        out_hbm.at[c].at[pl.ds(s * 640, 640)])

    return k(idx)


def _sc_mp128(h, src_rows, dst_rows):
    rows_per_w = IDX_ROWS // (NC * NS)

    @functools.partial(
        pl.kernel,
        out_type=jax.ShapeDtypeStruct((2, NP, 128), jnp.float32),
        mesh=_mesh,
        scratch_types=[
            pltpu.VMEM_SHARED((NP, 128), jnp.float32),
            pltpu.VMEM((_CH, 128), jnp.int32),
            pltpu.VMEM((_CH, 128), jnp.int32),
            pltpu.VMEM((128, 128), jnp.float32),
            pltpu.VMEM((128, 128), jnp.float32),
            pltpu.SemaphoreType.DMA,
            pltpu.SemaphoreType.DMA,
            pltpu.SemaphoreType.DMA,
            pltpu.SemaphoreType.DMA,
        ],
    )
    def k(h_hbm, src_hbm, dst_hbm, out_hbm, acc, sidx, didx, ra, rb,
          sa, sb, ssa, ssb):
        c = lax.axis_index("c")
        s = lax.axis_index("s")
        wid = s * NC + c
        _zero_rows(ra, 128, 128)
        for t in range(5):
            pltpu.sync_copy(ra, acc.at[pl.ds(s * 640 + t * 128, 128)])
        plsc.subcore_barrier()

        _mp_loop(h_hbm, src_hbm, dst_hbm, wid * rows_per_w, rows_per_w,
                 acc, sidx, didx, ra, rb, sa, sb, ssa, ssb)

        plsc.subcore_barrier()
        pltpu.sync_copy(acc.at[pl.ds(s * 640, 640)],
                        out_hbm.at[c].at[pl.ds(s * 640, 640)])

    return k(h, src_rows, dst_rows)


def _sc_mp256(hcat, src2, dst_rows):
    rows_per_sub = IDX_ROWS // NS

    @functools.partial(
        pl.kernel,
        out_type=jax.ShapeDtypeStruct((2, NP, 128), jnp.float32),
        mesh=_mesh,
        scratch_types=[
            pltpu.VMEM_SHARED((NP, 128), jnp.float32),
            pltpu.VMEM((_CH, 128), jnp.int32),
            pltpu.VMEM((_CH, 128), jnp.int32),
            pltpu.VMEM((128, 128), jnp.float32),
            pltpu.VMEM((128, 128), jnp.float32),
            pltpu.SemaphoreType.DMA,
            pltpu.SemaphoreType.DMA,
            pltpu.SemaphoreType.DMA,
            pltpu.SemaphoreType.DMA,
        ],
    )
    def k(h_hbm, src_hbm, dst_hbm, out_hbm, acc, sidx, didx, ra, rb,
          sa, sb, ssa, ssb):
        c = lax.axis_index("c")
        s = lax.axis_index("s")
        _zero_rows(ra, 128, 128)
        for t in range(5):
            pltpu.sync_copy(ra, acc.at[pl.ds(s * 640 + t * 128, 128)])
        plsc.subcore_barrier()

        _mp_loop(h_hbm, src_hbm.at[c], dst_hbm, s * rows_per_sub,
                 rows_per_sub, acc, sidx, didx, ra, rb, sa, sb, ssa, ssb)

        plsc.subcore_barrier()
        pltpu.sync_copy(acc.at[pl.ds(s * 640, 640)],
                        out_hbm.at[c].at[pl.ds(s * 640, 640)])

    return k(hcat, src2, dst_rows)


_GRID = (NP // BR,)


def _full2(shape):
    return pl.BlockSpec(shape, lambda i: tuple(0 for _ in shape))


def _rows2(cols):
    return pl.BlockSpec((BR, cols), lambda i: (i, 0))


def _rows3(cols):
    return pl.BlockSpec((2, BR, cols), lambda i: (0, i, 0))


def _stage0(degs, x_pad):
    def body(deg_ref, x_ref, s_ref, d_ref, xs_ref):
        dg = deg_ref[...]
        s_v = lax.rsqrt(jnp.maximum(dg[0], 1.0))
        d_v = lax.rsqrt(jnp.maximum(dg[1], 1.0))
        s_ref[...] = s_v[:, :16]
        d_ref[...] = d_v[:, :16]
        xs_ref[...] = x_ref[...] * s_v[:, :1]

    return pl.pallas_call(
        body,
        grid=_GRID,
        in_specs=[_rows3(128), _rows2(128)],
        out_specs=[_rows2(16), _rows2(16), _rows2(128)],
        out_shape=[
            jax.ShapeDtypeStruct((NP, 16), jnp.float32),
            jax.ShapeDtypeStruct((NP, 16), jnp.float32),
            jax.ShapeDtypeStruct((NP, 128), jnp.float32),
        ],
    )(degs, x_pad)


def _stage1(P1, d, W1, b1, W2, s):
    def body(p_ref, d_ref, w1_ref, b1_ref, w2_ref, s_ref, o_ref):
        a = (p_ref[0] + p_ref[1]) * d_ref[:, :1]
        h1 = jnp.dot(a, w1_ref[...], preferred_element_type=jnp.float32)
        h1 = jnp.maximum(h1 + b1_ref[...], 0.0)
        y2 = jnp.dot(h1, w2_ref[...], preferred_element_type=jnp.float32)
        y2 = y2 * s_ref[:, :1]
        o_ref[0] = y2[:, :128]
        o_ref[1] = y2[:, 128:]

    return pl.pallas_call(
        body,
        grid=_GRID,
        in_specs=[_rows3(128), _rows2(16), _full2((128, 800)),
                  _full2((1, 800)), _full2((800, 256)), _rows2(16)],
        out_specs=_rows3(128),
        out_shape=jax.ShapeDtypeStruct((2, NP, 128), jnp.float32),
    )(P1, d, W1, b1, W2, s)


def _stage2(P2, d, b2, s):
    def body(p_ref, d_ref, b2_ref, s_ref, enc_ref, t3_ref):
        dv = d_ref[:, :1]
        h0 = p_ref[0] * dv + b2_ref[:, :128]
        h1 = p_ref[1] * dv + b2_ref[:, 128:]
        enc_ref[...] = jax.nn.sigmoid(jnp.concatenate([h0, h1], axis=1))
        sv = s_ref[:, :1]
        t3_ref[0] = h0 * sv
        t3_ref[1] = h1 * sv

    return pl.pallas_call(
        body,
        grid=_GRID,
        in_specs=[_rows3(128), _rows2(16), _full2((1, 256)), _rows2(16)],
        out_specs=[_rows2(256), _rows3(128)],
        out_shape=[
            jax.ShapeDtypeStruct((NP, 256), jnp.float32),
            jax.ShapeDtypeStruct((2, NP, 128), jnp.float32),
        ],
    )(P2, d, b2, s)


def _stage3(P3, d, W3, b3, W4, s):
    def body(p_ref, d_ref, w3_ref, b3_ref, w4_ref, s_ref, o_ref):
        dv = d_ref[:, :1]
        a = jnp.concatenate([p_ref[0] * dv, p_ref[1] * dv], axis=1)
        g3 = jnp.dot(a, w3_ref[...], preferred_element_type=jnp.float32)
        g3 = jnp.maximum(g3 + b3_ref[...], 0.0)
        y4 = jnp.dot(g3, w4_ref[...], preferred_element_type=jnp.float32)
        o_ref[...] = y4 * s_ref[:, :1]

    return pl.pallas_call(
        body,
        grid=_GRID,
        in_specs=[_rows3(128), _rows2(16), _full2((256, 800)),
                  _full2((1, 800)), _full2((800, 128)), _rows2(16)],
        out_specs=_rows2(128),
        out_shape=jax.ShapeDtypeStruct((NP, 128), jnp.float32),
    )(P3, d, W3, b3, W4, s)


def _stage4(P4, d, b4):
    def body(p_ref, d_ref, b4_ref, o_ref):
        o_ref[...] = jax.nn.sigmoid(
            (p_ref[0] + p_ref[1]) * d_ref[:, :1] + b4_ref[...])

    return pl.pallas_call(
        body,
        grid=_GRID,
        in_specs=[_rows3(128), _rows2(16), _full2((1, 128))],
        out_specs=_rows2(128),
        out_shape=jax.ShapeDtypeStruct((NP, 128), jnp.float32),
    )(P4, d, b4)


def kernel(x, edge_index, W1, b1, W2, b2, W3, b3, W4, b4):
    src = edge_index[0]
    dst = edge_index[1]
    pad = jnp.full((EP - E,), N, jnp.int32)
    src_rows = jnp.concatenate([src, pad]).reshape(IDX_ROWS, 128)
    dst_rows = jnp.concatenate([dst, pad]).reshape(IDX_ROWS, 128)
    idx_stack = jnp.stack([src_rows, dst_rows])
    src2 = jnp.stack([src_rows, src_rows + NP])
    x_pad = jnp.pad(x, ((0, NP - N), (0, 0)))
    b1r = b1.reshape(1, -1)
    b2r = b2.reshape(1, -1)
    b3r = b3.reshape(1, -1)
    b4r = b4.reshape(1, -1)

    degs = _sc_degrees(idx_stack)
    s, d, xs = _stage0(degs, x_pad)
    P1 = _sc_mp128(xs, src_rows, dst_rows)
    y2s = _stage1(P1, d, W1, b1r, W2, s)
    P2 = _sc_mp256(y2s.reshape(2 * NP, 128), src2, dst_rows)
    enc_full, t3 = _stage2(P2, d, b2r, s)
    P3 = _sc_mp256(t3.reshape(2 * NP, 128), src2, dst_rows)
    y4s = _stage3(P3, d, W3, b3r, W4, s)
    P4 = _sc_mp128(y4s, src_rows, dst_rows)
    dec_full = _stage4(P4, d, b4r)
    return (enc_full[:N], dec_full[:N])


# speedup vs baseline: 2.3069x; 2.3069x over previous
import functools

import jax
import jax.numpy as jnp
from jax import lax
from jax.experimental import pallas as pl
from jax.experimental.pallas import tpu as pltpu
from jax.experimental.pallas import tpu_sc as plsc

N = 10000
NP = 10240
E = 320000
EP = 327680
IDX_ROWS = EP // 128
NC = 2
NS = 16
BR = 1024

_mesh = plsc.VectorSubcoreMesh(core_axis_name="c", subcore_axis_name="s")


def _zero_rows(buf, nrows, ncols):
    @pl.loop(0, nrows)
    def _(r):
        @pl.loop(0, ncols, step=16)
        def _(cc):
            buf[r, pl.ds(cc, 16)] = jnp.zeros((16,), jnp.float32)


_CH = 40


def _mp_loop(h_hbm, src_ref, dst_ref, base, rows_per, acc,
             sidx, didx, ra, rb, sa, sb, ssa, ssb):
    @pl.loop(0, rows_per, step=_CH)
    def _(ch):
        pltpu.sync_copy(src_ref.at[pl.ds(base + ch, _CH)], sidx)
        pltpu.sync_copy(dst_ref.at[pl.ds(base + ch, _CH)], didx)
        pltpu.async_copy(h_hbm.at[sidx.at[0]], ra, sa)
        pltpu.async_copy(h_hbm.at[sidx.at[1]], rb, sb)

        @pl.loop(0, _CH, step=2)
        def _(jj):
            pltpu.make_async_copy(h_hbm.at[sidx.at[jj]], ra, sa).wait()
            pltpu.async_copy(ra, acc.at[didx.at[jj]], ssa, add=True)
            pltpu.make_async_copy(h_hbm.at[sidx.at[jj + 1]], rb, sb).wait()
            pltpu.async_copy(rb, acc.at[didx.at[jj + 1]], ssb, add=True)
            pltpu.make_async_copy(ra, acc.at[didx.at[jj]], ssa).wait()

            @pl.when(jj + 2 < _CH)
            def _():
                pltpu.async_copy(h_hbm.at[sidx.at[jj + 2]], ra, sa)

            pltpu.make_async_copy(rb, acc.at[didx.at[jj + 1]], ssb).wait()

            @pl.when(jj + 3 < _CH)
            def _():
                pltpu.async_copy(h_hbm.at[sidx.at[jj + 3]], rb, sb)


def _sc_degrees(idx):
    rows_per_sub = IDX_ROWS // NS

    @functools.partial(
        pl.kernel,
        out_type=jax.ShapeDtypeStruct((2, NP, 128), jnp.float32),
        mesh=_mesh,
        scratch_types=[
            pltpu.VMEM_SHARED((NP, 128), jnp.float32),
            pltpu.VMEM((128, 128), jnp.float32),
            pltpu.VMEM((16, 128), jnp.int32),
        ],
    )
    def k(idx_hbm, out_hbm, acc, ones, didx):
        c = lax.axis_index("c")
        s = lax.axis_index("s")
        _zero_rows(ones, 128, 128)
        for t in range(5):
            pltpu.sync_copy(ones, acc.at[pl.ds(s * 640 + t * 128, 128)])

        @pl.loop(0, 128)
        def _(r):
            @pl.loop(0, 128, step=16)
            def _(cc):
                ones[r, pl.ds(cc, 16)] = jnp.ones((16,), jnp.float32)

        plsc.subcore_barrier()

        @pl.loop(0, rows_per_sub, step=16)
        def _(ch):
            pltpu.sync_copy(
                idx_hbm.at[c].at[pl.ds(s * rows_per_sub + ch, 16)], didx)

            @pl.loop(0, 16)
            def _(j):
                pltpu.sync_copy(ones, acc.at[didx.at[j]], add=True)

        plsc.subcore_barrier()
        pltpu.sync_copy(acc.at[pl.ds(s * 640, 640)],
                        out_hbm.at[c].at[pl.ds(s * 640, 640)])

    return k(idx)


def _sc_mp128(h, src_rows, dst_rows):
    rows_per_w = IDX_ROWS // (NC * NS)

    @functools.partial(
        pl.kernel,
        out_type=jax.ShapeDtypeStruct((2, NP, 128), jnp.float32),
        mesh=_mesh,
        scratch_types=[
            pltpu.VMEM_SHARED((NP, 128), jnp.float32),
            pltpu.VMEM((_CH, 128), jnp.int32),
            pltpu.VMEM((_CH, 128), jnp.int32),
            pltpu.VMEM((128, 128), jnp.float32),
            pltpu.VMEM((128, 128), jnp.float32),
            pltpu.SemaphoreType.DMA,
            pltpu.SemaphoreType.DMA,
            pltpu.SemaphoreType.DMA,
            pltpu.SemaphoreType.DMA,
        ],
    )
    def k(h_hbm, src_hbm, dst_hbm, out_hbm, acc, sidx, didx, ra, rb,
          sa, sb, ssa, ssb):
        c = lax.axis_index("c")
        s = lax.axis_index("s")
        wid = s * NC + c
        _zero_rows(ra, 128, 128)
        for t in range(5):
            pltpu.sync_copy(ra, acc.at[pl.ds(s * 640 + t * 128, 128)])
        plsc.subcore_barrier()

        _mp_loop(h_hbm, src_hbm, dst_hbm, wid * rows_per_w, rows_per_w,
                 acc, sidx, didx, ra, rb, sa, sb, ssa, ssb)

        plsc.subcore_barrier()
        pltpu.sync_copy(acc.at[pl.ds(s * 640, 640)],
                        out_hbm.at[c].at[pl.ds(s * 640, 640)])

    return k(h, src_rows, dst_rows)


def _sc_mp256(hcat, src2, dst_rows):
    rows_per_sub = IDX_ROWS // NS

    @functools.partial(
        pl.kernel,
        out_type=jax.ShapeDtypeStruct((2, NP, 128), jnp.float32),
        mesh=_mesh,
        scratch_types=[
            pltpu.VMEM_SHARED((NP, 128), jnp.float32),
            pltpu.VMEM((_CH, 128), jnp.int32),
            pltpu.VMEM((_CH, 128), jnp.int32),
            pltpu.VMEM((128, 128), jnp.float32),
            pltpu.VMEM((128, 128), jnp.float32),
            pltpu.SemaphoreType.DMA,
            pltpu.SemaphoreType.DMA,
            pltpu.SemaphoreType.DMA,
            pltpu.SemaphoreType.DMA,
        ],
    )
    def k(h_hbm, src_hbm, dst_hbm, out_hbm, acc, sidx, didx, ra, rb,
          sa, sb, ssa, ssb):
        c = lax.axis_index("c")
        s = lax.axis_index("s")
        _zero_rows(ra, 128, 128)
        for t in range(5):
            pltpu.sync_copy(ra, acc.at[pl.ds(s * 640 + t * 128, 128)])
        plsc.subcore_barrier()

        _mp_loop(h_hbm, src_hbm.at[c], dst_hbm, s * rows_per_sub,
                 rows_per_sub, acc, sidx, didx, ra, rb, sa, sb, ssa, ssb)

        plsc.subcore_barrier()
        pltpu.sync_copy(acc.at[pl.ds(s * 640, 640)],
                        out_hbm.at[c].at[pl.ds(s * 640, 640)])

    return k(hcat, src2, dst_rows)


_GRID = (NP // BR,)


def _full2(shape):
    return pl.BlockSpec(shape, lambda i: tuple(0 for _ in shape))


def _rows2(cols):
    return pl.BlockSpec((BR, cols), lambda i: (i, 0))


def _rows3(cols):
    return pl.BlockSpec((2, BR, cols), lambda i: (0, i, 0))


def _stage0(degs, x_pad):
    def body(deg_ref, x_ref, s_ref, d_ref, xs_ref):
        dg = deg_ref[...]
        s_v = lax.rsqrt(jnp.maximum(dg[0], 1.0))
        d_v = lax.rsqrt(jnp.maximum(dg[1], 1.0))
        s_ref[...] = s_v[:, :16]
        d_ref[...] = d_v[:, :16]
        xs_ref[...] = x_ref[...] * s_v[:, :1]

    return pl.pallas_call(
        body,
        grid=_GRID,
        in_specs=[_rows3(128), _rows2(128)],
        out_specs=[_rows2(16), _rows2(16), _rows2(128)],
        out_shape=[
            jax.ShapeDtypeStruct((NP, 16), jnp.float32),
            jax.ShapeDtypeStruct((NP, 16), jnp.float32),
            jax.ShapeDtypeStruct((NP, 128), jnp.float32),
        ],
    )(degs, x_pad)


def _stage1(P1, d, W1, b1, W2, s):
    def body(p_ref, d_ref, w1_ref, b1_ref, w2_ref, s_ref, o_ref):
        a = (p_ref[0] + p_ref[1]) * d_ref[:, :1]
        h1 = jnp.dot(a, w1_ref[...], preferred_element_type=jnp.float32)
        h1 = jnp.maximum(h1 + b1_ref[...], 0.0)
        y2 = jnp.dot(h1, w2_ref[...], preferred_element_type=jnp.float32)
        y2 = y2 * s_ref[:, :1]
        o_ref[0] = y2[:, :128]
        o_ref[1] = y2[:, 128:]

    return pl.pallas_call(
        body,
        grid=_GRID,
        in_specs=[_rows3(128), _rows2(16), _full2((128, 800)),
                  _full2((1, 800)), _full2((800, 256)), _rows2(16)],
        out_specs=_rows3(128),
        out_shape=jax.ShapeDtypeStruct((2, NP, 128), jnp.float32),
    )(P1, d, W1, b1, W2, s)


def _stage2(P2, d, b2, s):
    def body(p_ref, d_ref, b2_ref, s_ref, enc_ref, t3_ref):
        dv = d_ref[:, :1]
        h0 = p_ref[0] * dv + b2_ref[:, :128]
        h1 = p_ref[1] * dv + b2_ref[:, 128:]
        enc_ref[...] = jax.nn.sigmoid(jnp.concatenate([h0, h1], axis=1))
        sv = s_ref[:, :1]
        t3_ref[0] = h0 * sv
        t3_ref[1] = h1 * sv

    return pl.pallas_call(
        body,
        grid=_GRID,
        in_specs=[_rows3(128), _rows2(16), _full2((1, 256)), _rows2(16)],
        out_specs=[_rows2(256), _rows3(128)],
        out_shape=[
            jax.ShapeDtypeStruct((NP, 256), jnp.float32),
            jax.ShapeDtypeStruct((2, NP, 128), jnp.float32),
        ],
    )(P2, d, b2, s)


def _stage3(P3, d, W3, b3, W4, s):
    def body(p_ref, d_ref, w3_ref, b3_ref, w4_ref, s_ref, o_ref):
        dv = d_ref[:, :1]
        a = jnp.concatenate([p_ref[0] * dv, p_ref[1] * dv], axis=1)
        g3 = jnp.dot(a, w3_ref[...], preferred_element_type=jnp.float32)
        g3 = jnp.maximum(g3 + b3_ref[...], 0.0)
        y4 = jnp.dot(g3, w4_ref[...], preferred_element_type=jnp.float32)
        o_ref[...] = y4 * s_ref[:, :1]

    return pl.pallas_call(
        body,
        grid=_GRID,
        in_specs=[_rows3(128), _rows2(16), _full2((256, 800)),
                  _full2((1, 800)), _full2((800, 128)), _rows2(16)],
        out_specs=_rows2(128),
        out_shape=jax.ShapeDtypeStruct((NP, 128), jnp.float32),
    )(P3, d, W3, b3, W4, s)


def _stage4(P4, d, b4):
    def body(p_ref, d_ref, b4_ref, o_ref):
        o_ref[...] = jax.nn.sigmoid(
            (p_ref[0] + p_ref[1]) * d_ref[:, :1] + b4_ref[...])

    return pl.pallas_call(
        body,
        grid=_GRID,
        in_specs=[_rows3(128), _rows2(16), _full2((1, 128))],
        out_specs=_rows2(128),
        out_shape=jax.ShapeDtypeStruct((NP, 128), jnp.float32),
    )(P4, d, b4)


def kernel(x, edge_index, W1, b1, W2, b2, W3, b3, W4, b4):
    src = edge_index[0]
    dst = edge_index[1]
    pad = N + jnp.arange(EP - E, dtype=jnp.int32) % (NP - N)
    src_rows = jnp.concatenate([src, pad]).reshape(IDX_ROWS, 128)
    dst_rows = jnp.concatenate([dst, pad]).reshape(IDX_ROWS, 128)
    idx_stack = jnp.stack([src_rows, dst_rows])
    src2 = jnp.stack([src_rows, src_rows + NP])
    x_pad = jnp.pad(x, ((0, NP - N), (0, 0)))
    b1r = b1.reshape(1, -1)
    b2r = b2.reshape(1, -1)
    b3r = b3.reshape(1, -1)
    b4r = b4.reshape(1, -1)

    degs = _sc_degrees(idx_stack)
    s, d, xs = _stage0(degs, x_pad)
    P1 = _sc_mp128(xs, src_rows, dst_rows)
    y2s = _stage1(P1, d, W1, b1r, W2, s)
    P2 = _sc_mp256(y2s.reshape(2 * NP, 128), src2, dst_rows)
    enc_full, t3 = _stage2(P2, d, b2r, s)
    P3 = _sc_mp256(t3.reshape(2 * NP, 128), src2, dst_rows)
    y4s = _stage3(P3, d, W3, b3r, W4, s)
    P4 = _sc_mp128(y4s, src_rows, dst_rows)
    dec_full = _stage4(P4, d, b4r)
    return (enc_full[:N], dec_full[:N])
